# baseline (speedup 1.0000x reference)
import functools

import jax
import jax.numpy as jnp
from jax import lax
from jax.experimental import pallas as pl
from jax.experimental.pallas import tpu as pltpu
from jax.experimental.pallas import tpu_sc as plsc

N = 100000
E = 1600000
F = 32

NC = 2
NS = 16
L = 16

NP = 100352
RPT = NP // NS
EP = 1638400
ER = EP // 128
KB = 8
RT_FULL = ER // NS
RT_HALF = ER // (NS * NC)

BLK = 2048


def _mesh():
  return plsc.VectorSubcoreMesh(
      core_axis_name="c", subcore_axis_name="s", num_cores=NC,
      num_subcores=NS)


def _rsqrt16(x):
  i = lax.bitcast_convert_type(x, jnp.int32)
  i = jnp.int32(0x5F3759DF) - lax.shift_right_arithmetic(i, 1)
  y = lax.bitcast_convert_type(i, jnp.float32)
  for _ in range(3):
    y = y * (jnp.float32(1.5) - jnp.float32(0.5) * x * y * y)
  return y


def _k1_body(rowp, colp, valp, wp, zn,
             prop_o, x1p_o, pdiag_o,
             s_acc, c_acc, dis_sh, x1_acc,
             rba, vba, wba, rbb, vbb, wbb,
             aba, abb, cba, cbb, pba, pbb, grs, gcs,
             sbuf, cbuf, disbuf, pdbuf,
             isema, isemb, scsa, scsb, gsem, hsem, osema, osemb):
  cid = lax.axis_index("c")
  tid = lax.axis_index("s")
  nbase = tid * RPT

  pltpu.sync_copy(zn.at[pl.ds(0, RPT)], disbuf)
  pltpu.sync_copy(disbuf, s_acc.at[pl.ds(nbase, RPT)])
  pltpu.sync_copy(disbuf, c_acc.at[pl.ds(nbase, RPT)])
  pltpu.sync_copy(disbuf, x1_acc.at[pl.ds(nbase, RPT)])
  plsc.subcore_barrier()

  tbase = tid * RT_FULL
  SA = RT_FULL // KB

  def a_start(bufs, rbase):
    rb, vb, wb, sem = bufs
    pltpu.async_copy(rowp.at[pl.ds(rbase, KB)], rb, sem)
    pltpu.async_copy(valp.at[pl.ds(rbase, KB)], vb, sem)
    pltpu.async_copy(wp.at[pl.ds(rbase, KB)], wb, sem)

  def a_wait(bufs, rbase):
    rb, vb, wb, sem = bufs
    pltpu.make_async_copy(rowp.at[pl.ds(rbase, KB)], rb, sem).wait()
    pltpu.make_async_copy(valp.at[pl.ds(rbase, KB)], vb, sem).wait()
    pltpu.make_async_copy(wp.at[pl.ds(rbase, KB)], wb, sem).wait()

  def a_block(bufs, ab, scs, rbase, nxt):
    rb, vb, wb, sem = bufs
    a_wait(bufs, rbase)
    for j in range(KB):
      for e in range(128 // L):
        ab[j, pl.ds(e * L, L)] = jnp.abs(vb[j, pl.ds(e * L, L)])
      pltpu.async_copy(ab.at[j], s_acc.at[rb.at[j]], scs, add=True)
      pltpu.async_copy(wb.at[j], c_acc.at[rb.at[j]], scs, add=True)
    for j in range(KB):
      pltpu.make_async_copy(ab.at[j], s_acc.at[rb.at[j]], scs).wait()
      pltpu.make_async_copy(wb.at[j], c_acc.at[rb.at[j]], scs).wait()

    @pl.when(nxt < SA)
    def _():
      a_start(bufs, tbase + nxt * KB)

  bufs_a = (rba, vba, wba, isema)
  bufs_b = (rbb, vbb, wbb, isemb)
  a_start(bufs_a, tbase)
  a_start(bufs_b, tbase + KB)

  def a_super(m, _):
    a_block(bufs_a, aba, scsa, tbase + 2 * m * KB, 2 * m + 2)
    a_block(bufs_b, abb, scsb, tbase + (2 * m + 1) * KB, 2 * m + 3)
    return 0
  lax.fori_loop(0, SA // 2, a_super, 0)
  plsc.subcore_barrier()

  pltpu.sync_copy(s_acc.at[pl.ds(nbase, RPT)], sbuf)
  pltpu.sync_copy(c_acc.at[pl.ds(nbase, RPT)], cbuf)

  def blkB(i, _):
    sl = pl.ds(i * L, L)
    s = sbuf[sl]
    c = cbuf[sl]
    am = s / jnp.maximum(c, jnp.float32(1.0))
    deg = s + am
    r = _rsqrt16(jnp.maximum(deg, jnp.float32(1e-30)))
    dis = jnp.where(deg > 0, r, jnp.float32(0.0))
    disbuf[sl] = dis
    pdbuf[sl] = dis * dis * am
    return 0
  lax.fori_loop(0, RPT // L, blkB, 0)
  pltpu.sync_copy(disbuf, dis_sh.at[pl.ds(nbase, RPT)])

  @pl.when(cid == 0)
  def _():
    pltpu.sync_copy(pdbuf, pdiag_o.at[pl.ds(nbase, RPT)])
  plsc.subcore_barrier()

  wid = tid * NC + cid
  wbase = wid * RT_HALF
  SC_ = RT_HALF // KB

  def c_start(bufs, rbase):
    rb, cb, vb, sem = bufs
    pltpu.async_copy(rowp.at[pl.ds(rbase, KB)], rb, sem)
    pltpu.async_copy(colp.at[pl.ds(rbase, KB)], cb, sem)
    pltpu.async_copy(valp.at[pl.ds(rbase, KB)], vb, sem)

  def c_wait(bufs, rbase):
    rb, cb, vb, sem = bufs
    pltpu.make_async_copy(rowp.at[pl.ds(rbase, KB)], rb, sem).wait()
    pltpu.make_async_copy(colp.at[pl.ds(rbase, KB)], cb, sem).wait()
    pltpu.make_async_copy(valp.at[pl.ds(rbase, KB)], vb, sem).wait()

  def c_block(bufs, pb, scs, osem, rbase, nxt, first):
    rb, cb, vb, sem = bufs
    c_wait(bufs, rbase)
    for j in range(KB):
      pltpu.async_copy(dis_sh.at[rb.at[j]], grs.at[j], gsem.at[j])
      pltpu.async_copy(dis_sh.at[cb.at[j]], gcs.at[j], hsem.at[j])
    if not first:
      pltpu.make_async_copy(pb, prop_o.at[pl.ds(rbase, KB)], osem).wait()
    for j in range(KB):
      pltpu.make_async_copy(dis_sh.at[rb.at[j]], grs.at[j], gsem.at[j]).wait()
      pltpu.make_async_copy(dis_sh.at[cb.at[j]], gcs.at[j], hsem.at[j]).wait()
      for e in range(128 // L):
        sl = pl.ds(e * L, L)
        pb[j, sl] = grs[j, sl] * vb[j, sl] * gcs[j, sl]
      pltpu.async_copy(pb.at[j], x1_acc.at[cb.at[j]], scs, add=True)
    for j in range(KB):
      pltpu.make_async_copy(pb.at[j], x1_acc.at[cb.at[j]], scs).wait()
    pltpu.async_copy(pb, prop_o.at[pl.ds(rbase, KB)], osem)

    @pl.when(nxt < SC_)
    def _():
      c_start(bufs, wbase + nxt * KB)

  cbufs_a = (rba, cba, vba, isema)
  cbufs_b = (rbb, cbb, vbb, isemb)
  c_start(cbufs_a, wbase)
  c_start(cbufs_b, wbase + KB)

  c_block(cbufs_a, pba, scsa, osema, wbase, 2, True)
  c_block(cbufs_b, pbb, scsb, osemb, wbase + KB, 3, True)

  def c_super1(m, _):
    c_block(cbufs_a, pba, scsa, osema, wbase + 2 * (m + 1) * KB, 2 * m + 4,
            False)
    c_block(cbufs_b, pbb, scsb, osemb, wbase + (2 * (m + 1) + 1) * KB,
            2 * m + 5, False)
    return 0
  lax.fori_loop(0, SC_ // 2 - 1, c_super1, 0)

  pltpu.make_async_copy(pba, prop_o.at[pl.ds(0, KB)], osema).wait()
  pltpu.make_async_copy(pbb, prop_o.at[pl.ds(0, KB)], osemb).wait()
  plsc.subcore_barrier()

  pltpu.sync_copy(x1_acc.at[pl.ds(nbase, RPT)],
                  x1p_o.at[cid, pl.ds(nbase, RPT)])


def _run_k1(rowp, colp, valp, wp, zn):
  f32 = jnp.float32
  out_type = (
      jax.ShapeDtypeStruct((ER, 128), f32),
      jax.ShapeDtypeStruct((NC, NP), f32),
      jax.ShapeDtypeStruct((NP,), f32),
  )
  scratch = [
      pltpu.VMEM_SHARED((NP,), f32),
      pltpu.VMEM_SHARED((NP,), f32),
      pltpu.VMEM_SHARED((NP,), f32),
      pltpu.VMEM_SHARED((NP,), f32),
      pltpu.VMEM((KB, 128), jnp.int32),
      pltpu.VMEM((KB, 128), f32),
      pltpu.VMEM((KB, 128), f32),
      pltpu.VMEM((KB, 128), jnp.int32),
      pltpu.VMEM((KB, 128), f32),
      pltpu.VMEM((KB, 128), f32),
      pltpu.VMEM((KB, 128), f32),
      pltpu.VMEM((KB, 128), f32),
      pltpu.VMEM((KB, 128), jnp.int32),
      pltpu.VMEM((KB, 128), jnp.int32),
      pltpu.VMEM((KB, 128), f32),
      pltpu.VMEM((KB, 128), f32),
      pltpu.VMEM((KB, 128), f32),
      pltpu.VMEM((KB, 128), f32),
      pltpu.VMEM((RPT,), f32),
      pltpu.VMEM((RPT,), f32),
      pltpu.VMEM((RPT,), f32),
      pltpu.VMEM((RPT,), f32),
      pltpu.SemaphoreType.DMA,
      pltpu.SemaphoreType.DMA,
      pltpu.SemaphoreType.DMA,
      pltpu.SemaphoreType.DMA,
      pltpu.SemaphoreType.DMA((KB,)),
      pltpu.SemaphoreType.DMA((KB,)),
      pltpu.SemaphoreType.DMA,
      pltpu.SemaphoreType.DMA,
  ]
  k = pl.kernel(_k1_body, out_type=out_type, mesh=_mesh(),
                scratch_types=scratch,
                name="sc_k1_stats_prop")
  return k(rowp, colp, valp, wp, zn)


def _conv_body(nsl, passes, rowp, colp, propp, ys, init,
               agg_o,
               acc, rba, cba, pba, rbb, cbb, pbb, gbufs,
               gsem, ssem, isema, isemb):
  cid = lax.axis_index("c")
  tid = lax.axis_index("s")
  nbase = tid * RPT
  tbase = tid * RT_FULL
  SB = RT_FULL // (2 * KB)

  bufs_a = (rba, cba, pba, isema)
  bufs_b = (rbb, cbb, pbb, isemb)

  def idx_start(bufs, rbase):
    rb, cb, pb, sem = bufs
    pltpu.async_copy(rowp.at[pl.ds(rbase, KB)], rb, sem)
    pltpu.async_copy(colp.at[pl.ds(rbase, KB)], cb, sem)
    pltpu.async_copy(propp.at[pl.ds(rbase, KB)], pb, sem)

  def idx_wait(bufs, rbase):
    rb, cb, pb, sem = bufs
    pltpu.make_async_copy(rowp.at[pl.ds(rbase, KB)], rb, sem).wait()
    pltpu.make_async_copy(colp.at[pl.ds(rbase, KB)], cb, sem).wait()
    pltpu.make_async_copy(propp.at[pl.ds(rbase, KB)], pb, sem).wait()

  def s_wait(cb, j):
    pltpu.make_async_copy(gbufs.at[j], acc.at[cb.at[j]], ssem.at[j]).wait()

  def half(ysl, bufs, rbase, first):
    rb, cb, pb, sem = bufs
    idx_wait(bufs, rbase)
    for j in range(KB):
      if not first:
        s_wait(cb, j)
      pltpu.async_copy(ysl.at[rb.at[j]], gbufs.at[j], gsem.at[j])
    for j in range(KB):
      pltpu.make_async_copy(ysl.at[rb.at[j]], gbufs.at[j], gsem.at[j]).wait()

      def scale(g, _):
        pv = pb[j, pl.ds(g * L, L)]
        for l in range(L):
          e = g * L + l
          gbufs[j, e] = gbufs[j, e] * pv[l]
        return 0
      lax.fori_loop(0, 128 // L, scale, 0)
      pltpu.async_copy(gbufs.at[j], acc.at[cb.at[j]], ssem.at[j], add=True)

  for p in range(passes):
    sid = cid * passes + p
    ysl = ys.at[sid]

    pltpu.sync_copy(init.at[sid, pl.ds(nbase, RPT)],
                    acc.at[pl.ds(nbase, RPT)])
    plsc.subcore_barrier()

    ra = lambda m: tbase + m * 2 * KB
    rb_ = lambda m: tbase + m * 2 * KB + KB

    idx_start(bufs_a, ra(0))
    idx_start(bufs_b, rb_(0))
    half(ysl, bufs_a, ra(0), first=True)
    idx_start(bufs_a, ra(1))

    def sblk(m, _):
      half(ysl, bufs_b, rb_(m), first=False)
      idx_start(bufs_b, rb_(m + 1))
      half(ysl, bufs_a, ra(m + 1), first=False)

      @pl.when(m + 1 < SB - 1)
      def _():
        idx_start(bufs_a, ra(m + 2))
      return 0
    lax.fori_loop(0, SB - 1, sblk, 0)
    half(ysl, bufs_b, rb_(SB - 1), first=False)

    for j in range(KB):
      s_wait(cbb, j)
    plsc.subcore_barrier()

    pltpu.sync_copy(acc.at[pl.ds(nbase, RPT)],
                    agg_o.at[sid, pl.ds(nbase, RPT)])
    plsc.subcore_barrier()


def _run_conv(nsl, passes, rowp, colp, propp, ys, init):
  f32 = jnp.float32
  scratch = [
      pltpu.VMEM_SHARED((NP, L), f32),
      pltpu.VMEM((KB, 128), jnp.int32),
      pltpu.VMEM((KB, 128), jnp.int32),
      pltpu.VMEM((KB, 128), f32),
      pltpu.VMEM((KB, 128), jnp.int32),
      pltpu.VMEM((KB, 128), jnp.int32),
      pltpu.VMEM((KB, 128), f32),
      pltpu.VMEM((KB, 128, L), f32),
      pltpu.SemaphoreType.DMA((KB,)),
      pltpu.SemaphoreType.DMA((KB,)),
      pltpu.SemaphoreType.DMA,
      pltpu.SemaphoreType.DMA,
  ]
  k = pl.kernel(functools.partial(_conv_body, nsl, passes),
                out_type=jax.ShapeDtypeStruct((nsl, NP, L), f32),
                mesh=_mesh(), scratch_types=scratch,
                compiler_params=pltpu.CompilerParams(use_tc_tiling_on_sc=False),
                name=f"sc_conv_{nsl}x16")
  return k(rowp, colp, propp, ys, init)


def _leaky(v):
  return jnp.where(v >= 0, v, jnp.float32(0.1) * v)


def _tc_d_body(x1p, pd, wl, bl, w2, b2, y2_o, init_o):
  x1 = x1p[0, :] + x1p[1, :] + pd[...]
  h = x1[:, None] * wl[0][None, :] + bl[0][None, :]
  h = _leaky(h)
  y2 = jnp.dot(h, w2[...], preferred_element_type=jnp.float32) + b2[0][None, :]
  pdc = pd[...][:, None]
  for k in range(2 * F // L):
    sl = y2[:, k * L:(k + 1) * L]
    y2_o[k] = sl
    init_o[k] = pdc * sl


def _run_tc_d(x1p, pdiag, W_lin, b_lin, W2, b2):
  f32 = jnp.float32
  nsl = 2 * F // L
  grid = (NP // BLK,)
  return pl.pallas_call(
      _tc_d_body,
      grid=grid,
      in_specs=[
          pl.BlockSpec((NC, BLK), lambda i: (0, i)),
          pl.BlockSpec((BLK,), lambda i: (i,)),
          pl.BlockSpec((1, F), lambda i: (0, 0)),
          pl.BlockSpec((1, F), lambda i: (0, 0)),
          pl.BlockSpec((F, 2 * F), lambda i: (0, 0)),
          pl.BlockSpec((1, 2 * F), lambda i: (0, 0)),
      ],
      out_specs=[
          pl.BlockSpec((nsl, BLK, L), lambda i: (0, i, 0)),
          pl.BlockSpec((nsl, BLK, L), lambda i: (0, i, 0)),
      ],
      out_shape=[
          jax.ShapeDtypeStruct((nsl, NP, L), f32),
          jax.ShapeDtypeStruct((nsl, NP, L), f32),
      ],
      name="tc_d_x1_to_y2",
  )(x1p, pdiag, W_lin.reshape(1, F), b_lin.reshape(1, F), W2,
    b2.reshape(1, 2 * F))


def _tc_f_body(agg, pd, bias2, w3, b3, y3_o, init_o):
  cat = jnp.concatenate([agg[k] for k in range(2 * F // L)], axis=1)
  z2 = _leaky(cat + bias2[0][None, :])
  y3 = jnp.dot(z2, w3[...], preferred_element_type=jnp.float32) + b3[0][None, :]
  pdc = pd[...][:, None]
  for k in range(F // L):
    sl = y3[:, k * L:(k + 1) * L]
    y3_o[k] = sl
    init_o[k] = pdc * sl


def _run_tc_f(agg2, pdiag, bias2, W3, b3):
  f32 = jnp.float32
  nin = 2 * F // L
  nout = F // L
  return pl.pallas_call(
      _tc_f_body,
      grid=(NP // BLK,),
      in_specs=[
          pl.BlockSpec((nin, BLK, L), lambda i: (0, i, 0)),
          pl.BlockSpec((BLK,), lambda i: (i,)),
          pl.BlockSpec((1, 2 * F), lambda i: (0, 0)),
          pl.BlockSpec((2 * F, F), lambda i: (0, 0)),
          pl.BlockSpec((1, F), lambda i: (0, 0)),
      ],
      out_specs=[
          pl.BlockSpec((nout, BLK, L), lambda i: (0, i, 0)),
          pl.BlockSpec((nout, BLK, L), lambda i: (0, i, 0)),
      ],
      out_shape=[
          jax.ShapeDtypeStruct((nout, NP, L), f32),
          jax.ShapeDtypeStruct((nout, NP, L), f32),
      ],
      name="tc_f_z2_to_y3",
  )(agg2, pdiag, bias2.reshape(1, 2 * F), W3, b3.reshape(1, F))


def _tc_h_body(agg, bias3, out_o):
  cat = jnp.concatenate([agg[k] for k in range(F // L)], axis=1)
  out_o[...] = _leaky(cat + bias3[0][None, :])


def _run_tc_h(agg3, bias3):
  return pl.pallas_call(
      _tc_h_body,
      grid=(NP // BLK,),
      in_specs=[
          pl.BlockSpec((F // L, BLK, L), lambda i: (0, i, 0)),
          pl.BlockSpec((1, F), lambda i: (0, 0)),
      ],
      out_specs=pl.BlockSpec((BLK, F), lambda i: (i, 0)),
      out_shape=jax.ShapeDtypeStruct((NP, F), jnp.float32),
      name="tc_h_final",
  )(agg3, bias3.reshape(1, F))


def kernel(edge_index, edge_val, W_lin, b_lin, W2, b2, bias2, W3, b3, bias3):
  f32 = jnp.float32
  row = edge_index[0]
  col = edge_index[1]
  padn = EP - E
  rowp = jnp.pad(row, (0, padn)).reshape(ER, 128)
  colp = jnp.pad(col, (0, padn)).reshape(ER, 128)
  valp = jnp.pad(edge_val, (0, padn)).reshape(ER, 128)
  wp = jnp.pad(jnp.ones((E,), f32), (0, padn)).reshape(ER, 128)
  zn = jnp.zeros((RPT,), f32)

  prop, x1p, pdiag = _run_k1(rowp, colp, valp, wp, zn)

  y2s, init2 = _run_tc_d(x1p, pdiag, W_lin, b_lin, W2, b2)
  agg2 = _run_conv(2 * F // L, 2, rowp, colp, prop, y2s, init2)

  y3s, init3 = _run_tc_f(agg2, pdiag, bias2, W3, b3)
  agg3 = _run_conv(F // L, 1, rowp, colp, prop, y3s, init3)

  out = _run_tc_h(agg3, bias3)
  return out[:N]


# speedup vs baseline: 1.4275x; 1.4275x over previous
import functools

import jax
import jax.numpy as jnp
from jax import lax
from jax.experimental import pallas as pl
from jax.experimental.pallas import tpu as pltpu
from jax.experimental.pallas import tpu_sc as plsc

N = 100000
E = 1600000
F = 32

NC = 2
NS = 16
L = 16

NP = 100352
RPT = NP // NS
EP = 1638400
ER = EP // 128
KB = 8
RT_FULL = ER // NS
RT_HALF = ER // (NS * NC)

BLK = 2048


def _mesh():
  return plsc.VectorSubcoreMesh(
      core_axis_name="c", subcore_axis_name="s", num_cores=NC,
      num_subcores=NS)


def _rsqrt16(x):
  i = lax.bitcast_convert_type(x, jnp.int32)
  i = jnp.int32(0x5F3759DF) - lax.shift_right_arithmetic(i, 1)
  y = lax.bitcast_convert_type(i, jnp.float32)
  for _ in range(3):
    y = y * (jnp.float32(1.5) - jnp.float32(0.5) * x * y * y)
  return y


def _k1_body(rowp, colp, valp, wp, zn,
             prop_o, x1p_o, pdiag_o,
             s_acc, c_acc, dis_sh, x1_acc,
             rba, vba, wba, rbb, vbb, wbb,
             aba, abb, cba, cbb, pba, pbb, grs, gcs,
             sbuf, cbuf, disbuf, pdbuf,
             isema, isemb, scsa, scsb, gsem, hsem, osema, osemb):
  cid = lax.axis_index("c")
  tid = lax.axis_index("s")
  nbase = tid * RPT

  pltpu.sync_copy(zn.at[pl.ds(0, RPT)], disbuf)
  pltpu.sync_copy(disbuf, s_acc.at[pl.ds(nbase, RPT)])
  pltpu.sync_copy(disbuf, c_acc.at[pl.ds(nbase, RPT)])
  pltpu.sync_copy(disbuf, x1_acc.at[pl.ds(nbase, RPT)])
  plsc.subcore_barrier()

  tbase = tid * RT_FULL
  SA = RT_FULL // KB

  def a_start(bufs, rbase):
    rb, vb, wb, sem = bufs
    pltpu.async_copy(rowp.at[pl.ds(rbase, KB)], rb, sem)
    pltpu.async_copy(valp.at[pl.ds(rbase, KB)], vb, sem)
    pltpu.async_copy(wp.at[pl.ds(rbase, KB)], wb, sem)

  def a_wait(bufs, rbase):
    rb, vb, wb, sem = bufs
    pltpu.make_async_copy(rowp.at[pl.ds(rbase, KB)], rb, sem).wait()
    pltpu.make_async_copy(valp.at[pl.ds(rbase, KB)], vb, sem).wait()
    pltpu.make_async_copy(wp.at[pl.ds(rbase, KB)], wb, sem).wait()

  def a_block(bufs, ab, scs, rbase, nxt):
    rb, vb, wb, sem = bufs
    a_wait(bufs, rbase)
    for j in range(KB):
      for e in range(128 // L):
        ab[j, pl.ds(e * L, L)] = jnp.abs(vb[j, pl.ds(e * L, L)])
      pltpu.async_copy(ab.at[j], s_acc.at[rb.at[j]], scs, add=True)
      pltpu.async_copy(wb.at[j], c_acc.at[rb.at[j]], scs, add=True)
    for j in range(KB):
      pltpu.make_async_copy(ab.at[j], s_acc.at[rb.at[j]], scs).wait()
      pltpu.make_async_copy(wb.at[j], c_acc.at[rb.at[j]], scs).wait()

    @pl.when(nxt < SA)
    def _():
      a_start(bufs, tbase + nxt * KB)

  bufs_a = (rba, vba, wba, isema)
  bufs_b = (rbb, vbb, wbb, isemb)
  a_start(bufs_a, tbase)
  a_start(bufs_b, tbase + KB)

  def a_super(m, _):
    a_block(bufs_a, aba, scsa, tbase + 2 * m * KB, 2 * m + 2)
    a_block(bufs_b, abb, scsb, tbase + (2 * m + 1) * KB, 2 * m + 3)
    return 0
  lax.fori_loop(0, SA // 2, a_super, 0)
  plsc.subcore_barrier()

  pltpu.sync_copy(s_acc.at[pl.ds(nbase, RPT)], sbuf)
  pltpu.sync_copy(c_acc.at[pl.ds(nbase, RPT)], cbuf)

  def blkB(i, _):
    sl = pl.ds(i * L, L)
    s = sbuf[sl]
    c = cbuf[sl]
    am = s / jnp.maximum(c, jnp.float32(1.0))
    deg = s + am
    r = _rsqrt16(jnp.maximum(deg, jnp.float32(1e-30)))
    dis = jnp.where(deg > 0, r, jnp.float32(0.0))
    disbuf[sl] = dis
    pdbuf[sl] = dis * dis * am
    return 0
  lax.fori_loop(0, RPT // L, blkB, 0)
  pltpu.sync_copy(disbuf, dis_sh.at[pl.ds(nbase, RPT)])

  @pl.when(cid == 0)
  def _():
    pltpu.sync_copy(pdbuf, pdiag_o.at[pl.ds(nbase, RPT)])
  plsc.subcore_barrier()

  wid = tid * NC + cid
  wbase = wid * RT_HALF
  SC_ = RT_HALF // KB

  def c_start(bufs, rbase):
    rb, cb, vb, sem = bufs
    pltpu.async_copy(rowp.at[pl.ds(rbase, KB)], rb, sem)
    pltpu.async_copy(colp.at[pl.ds(rbase, KB)], cb, sem)
    pltpu.async_copy(valp.at[pl.ds(rbase, KB)], vb, sem)

  def c_wait(bufs, rbase):
    rb, cb, vb, sem = bufs
    pltpu.make_async_copy(rowp.at[pl.ds(rbase, KB)], rb, sem).wait()
    pltpu.make_async_copy(colp.at[pl.ds(rbase, KB)], cb, sem).wait()
    pltpu.make_async_copy(valp.at[pl.ds(rbase, KB)], vb, sem).wait()

  def c_block(bufs, pb, scs, osem, rbase, nxt, first):
    rb, cb, vb, sem = bufs
    c_wait(bufs, rbase)
    for j in range(KB):
      pltpu.async_copy(dis_sh.at[rb.at[j]], grs.at[j], gsem.at[j])
      pltpu.async_copy(dis_sh.at[cb.at[j]], gcs.at[j], hsem.at[j])
    if not first:
      pltpu.make_async_copy(pb, prop_o.at[pl.ds(rbase, KB)], osem).wait()
    for j in range(KB):
      pltpu.make_async_copy(dis_sh.at[rb.at[j]], grs.at[j], gsem.at[j]).wait()
      pltpu.make_async_copy(dis_sh.at[cb.at[j]], gcs.at[j], hsem.at[j]).wait()
      for e in range(128 // L):
        sl = pl.ds(e * L, L)
        pb[j, sl] = grs[j, sl] * vb[j, sl] * gcs[j, sl]
      pltpu.async_copy(pb.at[j], x1_acc.at[cb.at[j]], scs, add=True)
    for j in range(KB):
      pltpu.make_async_copy(pb.at[j], x1_acc.at[cb.at[j]], scs).wait()
    pltpu.async_copy(pb, prop_o.at[pl.ds(rbase, KB)], osem)

    @pl.when(nxt < SC_)
    def _():
      c_start(bufs, wbase + nxt * KB)

  cbufs_a = (rba, cba, vba, isema)
  cbufs_b = (rbb, cbb, vbb, isemb)
  c_start(cbufs_a, wbase)
  c_start(cbufs_b, wbase + KB)

  c_block(cbufs_a, pba, scsa, osema, wbase, 2, True)
  c_block(cbufs_b, pbb, scsb, osemb, wbase + KB, 3, True)

  def c_super1(m, _):
    c_block(cbufs_a, pba, scsa, osema, wbase + 2 * (m + 1) * KB, 2 * m + 4,
            False)
    c_block(cbufs_b, pbb, scsb, osemb, wbase + (2 * (m + 1) + 1) * KB,
            2 * m + 5, False)
    return 0
  lax.fori_loop(0, SC_ // 2 - 1, c_super1, 0)

  pltpu.make_async_copy(pba, prop_o.at[pl.ds(0, KB)], osema).wait()
  pltpu.make_async_copy(pbb, prop_o.at[pl.ds(0, KB)], osemb).wait()
  plsc.subcore_barrier()

  pltpu.sync_copy(x1_acc.at[pl.ds(nbase, RPT)],
                  x1p_o.at[cid, pl.ds(nbase, RPT)])


def _run_k1(rowp, colp, valp, wp, zn):
  f32 = jnp.float32
  out_type = (
      jax.ShapeDtypeStruct((ER, 128), f32),
      jax.ShapeDtypeStruct((NC, NP), f32),
      jax.ShapeDtypeStruct((NP,), f32),
  )
  scratch = [
      pltpu.VMEM_SHARED((NP,), f32),
      pltpu.VMEM_SHARED((NP,), f32),
      pltpu.VMEM_SHARED((NP,), f32),
      pltpu.VMEM_SHARED((NP,), f32),
      pltpu.VMEM((KB, 128), jnp.int32),
      pltpu.VMEM((KB, 128), f32),
      pltpu.VMEM((KB, 128), f32),
      pltpu.VMEM((KB, 128), jnp.int32),
      pltpu.VMEM((KB, 128), f32),
      pltpu.VMEM((KB, 128), f32),
      pltpu.VMEM((KB, 128), f32),
      pltpu.VMEM((KB, 128), f32),
      pltpu.VMEM((KB, 128), jnp.int32),
      pltpu.VMEM((KB, 128), jnp.int32),
      pltpu.VMEM((KB, 128), f32),
      pltpu.VMEM((KB, 128), f32),
      pltpu.VMEM((KB, 128), f32),
      pltpu.VMEM((KB, 128), f32),
      pltpu.VMEM((RPT,), f32),
      pltpu.VMEM((RPT,), f32),
      pltpu.VMEM((RPT,), f32),
      pltpu.VMEM((RPT,), f32),
      pltpu.SemaphoreType.DMA,
      pltpu.SemaphoreType.DMA,
      pltpu.SemaphoreType.DMA,
      pltpu.SemaphoreType.DMA,
      pltpu.SemaphoreType.DMA((KB,)),
      pltpu.SemaphoreType.DMA((KB,)),
      pltpu.SemaphoreType.DMA,
      pltpu.SemaphoreType.DMA,
  ]
  k = pl.kernel(_k1_body, out_type=out_type, mesh=_mesh(),
                scratch_types=scratch,
                name="sc_k1_stats_prop")
  return k(rowp, colp, valp, wp, zn)


def _conv_body(nsl, passes, rowp, colp, propp, ys, init,
               agg_o,
               acc, rba, cba, pba, rbb, cbb, pbb, gbufs,
               gsem, ssem, isema, isemb):
  cid = lax.axis_index("c")
  tid = lax.axis_index("s")
  nbase = tid * RPT
  tbase = tid * RT_FULL
  SB = RT_FULL // (2 * KB)

  bufs_a = (rba, cba, pba, isema)
  bufs_b = (rbb, cbb, pbb, isemb)

  def idx_start(bufs, rbase):
    rb, cb, pb, sem = bufs
    pltpu.async_copy(rowp.at[pl.ds(rbase, KB)], rb, sem)
    pltpu.async_copy(colp.at[pl.ds(rbase, KB)], cb, sem)
    pltpu.async_copy(propp.at[pl.ds(rbase, KB)], pb, sem)

  def idx_wait(bufs, rbase):
    rb, cb, pb, sem = bufs
    pltpu.make_async_copy(rowp.at[pl.ds(rbase, KB)], rb, sem).wait()
    pltpu.make_async_copy(colp.at[pl.ds(rbase, KB)], cb, sem).wait()
    pltpu.make_async_copy(propp.at[pl.ds(rbase, KB)], pb, sem).wait()

  def s_wait(cb, j):
    pltpu.make_async_copy(gbufs.at[j], acc.at[cb.at[j]], ssem.at[j]).wait()

  def half(ysl, bufs, rbase, first):
    rb, cb, pb, sem = bufs
    idx_wait(bufs, rbase)
    for j in range(KB):
      if not first:
        s_wait(cb, j)
      pltpu.async_copy(ysl.at[rb.at[j]], gbufs.at[j], gsem.at[j])
    for j in range(KB):
      pltpu.make_async_copy(ysl.at[rb.at[j]], gbufs.at[j], gsem.at[j]).wait()

      def scale(g, _):
        pv = pb[j, pl.ds(g * L, L)]
        for l in range(L):
          e = g * L + l
          gbufs[j, e] = gbufs[j, e] * pv[l]
        return 0
      lax.fori_loop(0, 128 // L, scale, 0)
      pltpu.async_copy(gbufs.at[j], acc.at[cb.at[j]], ssem.at[j], add=True)

  for p in range(passes):
    sid = cid * passes + p
    ysl = ys.at[sid]

    pltpu.sync_copy(init.at[sid, pl.ds(nbase, RPT)],
                    acc.at[pl.ds(nbase, RPT)])
    plsc.subcore_barrier()

    ra = lambda m: tbase + m * 2 * KB
    rb_ = lambda m: tbase + m * 2 * KB + KB

    idx_start(bufs_a, ra(0))
    idx_start(bufs_b, rb_(0))
    half(ysl, bufs_a, ra(0), first=True)
    idx_start(bufs_a, ra(1))

    def sblk(m, _):
      half(ysl, bufs_b, rb_(m), first=False)
      idx_start(bufs_b, rb_(m + 1))
      half(ysl, bufs_a, ra(m + 1), first=False)

      @pl.when(m + 1 < SB - 1)
      def _():
        idx_start(bufs_a, ra(m + 2))
      return 0
    lax.fori_loop(0, SB - 1, sblk, 0)
    half(ysl, bufs_b, rb_(SB - 1), first=False)

    for j in range(KB):
      s_wait(cbb, j)
    plsc.subcore_barrier()

    pltpu.sync_copy(acc.at[pl.ds(nbase, RPT)],
                    agg_o.at[sid, pl.ds(nbase, RPT)])
    plsc.subcore_barrier()


def _run_conv(nsl, passes, rowp, colp, propp, ys, init):
  f32 = jnp.float32
  scratch = [
      pltpu.VMEM_SHARED((NP, L), f32),
      pltpu.VMEM((KB, 128), jnp.int32),
      pltpu.VMEM((KB, 128), jnp.int32),
      pltpu.VMEM((KB, 128), f32),
      pltpu.VMEM((KB, 128), jnp.int32),
      pltpu.VMEM((KB, 128), jnp.int32),
      pltpu.VMEM((KB, 128), f32),
      pltpu.VMEM((KB, 128, L), f32),
      pltpu.SemaphoreType.DMA((KB,)),
      pltpu.SemaphoreType.DMA((KB,)),
      pltpu.SemaphoreType.DMA,
      pltpu.SemaphoreType.DMA,
  ]
  k = pl.kernel(functools.partial(_conv_body, nsl, passes),
                out_type=jax.ShapeDtypeStruct((nsl, NP, L), f32),
                mesh=_mesh(), scratch_types=scratch,
                compiler_params=pltpu.CompilerParams(use_tc_tiling_on_sc=False),
                name=f"sc_conv_{nsl}x16")
  return k(rowp, colp, propp, ys, init)


def _leaky(v):
  return jnp.where(v >= 0, v, jnp.float32(0.1) * v)


def _tc_d_body(x1p, pd, wl, bl, w2, b2, y2_o, init_o):
  x1 = x1p[0, :] + x1p[1, :] + pd[...]
  h = x1[:, None] * wl[0][None, :] + bl[0][None, :]
  h = _leaky(h)
  y2 = jnp.dot(h, w2[...], preferred_element_type=jnp.float32) + b2[0][None, :]
  pdc = pd[...][:, None]
  for k in range(2 * F // L):
    sl = y2[:, k * L:(k + 1) * L]
    y2_o[k] = sl
    init_o[k] = pdc * sl


def _run_tc_d(x1p, pdiag, W_lin, b_lin, W2, b2):
  f32 = jnp.float32
  nsl = 2 * F // L
  grid = (NP // BLK,)
  return pl.pallas_call(
      _tc_d_body,
      grid=grid,
      in_specs=[
          pl.BlockSpec((NC, BLK), lambda i: (0, i)),
          pl.BlockSpec((BLK,), lambda i: (i,)),
          pl.BlockSpec((1, F), lambda i: (0, 0)),
          pl.BlockSpec((1, F), lambda i: (0, 0)),
          pl.BlockSpec((F, 2 * F), lambda i: (0, 0)),
          pl.BlockSpec((1, 2 * F), lambda i: (0, 0)),
      ],
      out_specs=[
          pl.BlockSpec((nsl, BLK, L), lambda i: (0, i, 0)),
          pl.BlockSpec((nsl, BLK, L), lambda i: (0, i, 0)),
      ],
      out_shape=[
          jax.ShapeDtypeStruct((nsl, NP, L), f32),
          jax.ShapeDtypeStruct((nsl, NP, L), f32),
      ],
      name="tc_d_x1_to_y2",
  )(x1p, pdiag, W_lin.reshape(1, F), b_lin.reshape(1, F), W2,
    b2.reshape(1, 2 * F))


def _tc_f_body(agg, pd, bias2, w3, b3, y3_o, init_o):
  cat = jnp.concatenate([agg[k] for k in range(2 * F // L)], axis=1)
  z2 = _leaky(cat + bias2[0][None, :])
  y3 = jnp.dot(z2, w3[...], preferred_element_type=jnp.float32) + b3[0][None, :]
  pdc = pd[...][:, None]
  for k in range(F // L):
    sl = y3[:, k * L:(k + 1) * L]
    y3_o[k] = sl
    init_o[k] = pdc * sl


def _run_tc_f(agg2, pdiag, bias2, W3, b3):
  f32 = jnp.float32
  nin = 2 * F // L
  nout = F // L
  return pl.pallas_call(
      _tc_f_body,
      grid=(NP // BLK,),
      in_specs=[
          pl.BlockSpec((nin, BLK, L), lambda i: (0, i, 0)),
          pl.BlockSpec((BLK,), lambda i: (i,)),
          pl.BlockSpec((1, 2 * F), lambda i: (0, 0)),
          pl.BlockSpec((2 * F, F), lambda i: (0, 0)),
          pl.BlockSpec((1, F), lambda i: (0, 0)),
      ],
      out_specs=[
          pl.BlockSpec((nout, BLK, L), lambda i: (0, i, 0)),
          pl.BlockSpec((nout, BLK, L), lambda i: (0, i, 0)),
      ],
      out_shape=[
          jax.ShapeDtypeStruct((nout, NP, L), f32),
          jax.ShapeDtypeStruct((nout, NP, L), f32),
      ],
      name="tc_f_z2_to_y3",
  )(agg2, pdiag, bias2.reshape(1, 2 * F), W3, b3.reshape(1, F))


def _tc_h_body(agg, bias3, out_o):
  cat = jnp.concatenate([agg[k] for k in range(F // L)], axis=1)
  out_o[...] = _leaky(cat + bias3[0][None, :])


def _run_tc_h(agg3, bias3):
  return pl.pallas_call(
      _tc_h_body,
      grid=(NP // BLK,),
      in_specs=[
          pl.BlockSpec((F // L, BLK, L), lambda i: (0, i, 0)),
          pl.BlockSpec((1, F), lambda i: (0, 0)),
      ],
      out_specs=pl.BlockSpec((BLK, F), lambda i: (i, 0)),
      out_shape=jax.ShapeDtypeStruct((NP, F), jnp.float32),
      name="tc_h_final",
  )(agg3, bias3.reshape(1, F))


def kernel(edge_index, edge_val, W_lin, b_lin, W2, b2, bias2, W3, b3, bias3):
  f32 = jnp.float32
  row = edge_index[0]
  col = edge_index[1]
  padn = EP - E
  pad_idx = jnp.arange(padn, dtype=jnp.int32) % NP
  rowp = jnp.concatenate([row, pad_idx]).reshape(ER, 128)
  colp = jnp.concatenate([col, pad_idx]).reshape(ER, 128)
  valp = jnp.pad(edge_val, (0, padn)).reshape(ER, 128)
  wp = jnp.pad(jnp.ones((E,), f32), (0, padn)).reshape(ER, 128)
  zn = jnp.zeros((RPT,), f32)

  prop, x1p, pdiag = _run_k1(rowp, colp, valp, wp, zn)

  y2s, init2 = _run_tc_d(x1p, pdiag, W_lin, b_lin, W2, b2)
  agg2 = _run_conv(2 * F // L, 2, rowp, colp, prop, y2s, init2)

  y3s, init3 = _run_tc_f(agg2, pdiag, bias2, W3, b3)
  agg3 = _run_conv(F // L, 1, rowp, colp, prop, y3s, init3)

  out = _run_tc_h(agg3, bias3)
  return out[:N]
